# baseline (speedup 1.0000x reference)
import functools

import jax
import jax.numpy as jnp
from jax import lax
from jax.experimental import pallas as pl
from jax.experimental.pallas import tpu as pltpu
from jax.experimental.pallas import tpu_sc as plsc

N = 10000
NPAD = 10240
E = 320000
NC = 2
NS = 16
NW = NC * NS
CHUNK = 80
NCHUNK = 128
IDXBLK = 16
NBLK = NCHUNK // IDXBLK
EPW = NCHUNK * CHUNK
EPAD = NW * EPW - E
RPS = NPAD // NS


def _seg_sum_partials(y, src3, dst3, zrows):
    D = y.shape[1]
    mesh = plsc.VectorSubcoreMesh(core_axis_name="c", subcore_axis_name="s")

    @functools.partial(
        pl.kernel,
        out_type=jax.ShapeDtypeStruct((NC, NPAD, D), jnp.float32),
        mesh=mesh,
        scratch_types=[
            pltpu.VMEM((IDXBLK, CHUNK), jnp.int32),
            pltpu.VMEM((IDXBLK, CHUNK), jnp.int32),
            pltpu.VMEM((CHUNK, D), jnp.float32),
            pltpu.VMEM((CHUNK, D), jnp.float32),
            pltpu.VMEM_SHARED((NPAD, D), jnp.float32),
            pltpu.SemaphoreType.DMA,
            pltpu.SemaphoreType.DMA,
        ],
    )
    def k(y_hbm, src_hbm, dst_hbm, z_hbm, out_hbm,
          src_v, dst_v, rows0, rows1, acc, sem0, sem1):
        cid = lax.axis_index("c")
        sid = lax.axis_index("s")
        wid = cid * NS + sid

        pltpu.sync_copy(z_hbm, acc.at[pl.ds(sid * RPS, RPS)])
        plsc.subcore_barrier()

        def scatter(ci, buf):
            pltpu.sync_copy(buf, acc.at[dst_v.at[ci]], add=True)

        @pl.loop(0, NBLK)
        def _(b):
            pltpu.sync_copy(src_hbm.at[wid, pl.ds(b * IDXBLK, IDXBLK)], src_v)
            pltpu.sync_copy(dst_hbm.at[wid, pl.ds(b * IDXBLK, IDXBLK)], dst_v)

            @pl.loop(0, IDXBLK)
            def _(ci):
                pltpu.sync_copy(y_hbm.at[src_v.at[ci]], rows0)
                scatter(ci, rows0)

        plsc.subcore_barrier()
        pltpu.sync_copy(acc.at[pl.ds(sid * RPS, RPS)],
                        out_hbm.at[cid, pl.ds(sid * RPS, RPS)])

    return k(y, src3, dst3, zrows)


def _matmul(x, W):
    n, K = x.shape
    Do = W.shape[1]
    BN = 1000

    def body(x_ref, w_ref, o_ref):
        o_ref[...] = jnp.dot(x_ref[...], w_ref[...],
                             preferred_element_type=jnp.float32)

    return pl.pallas_call(
        body,
        grid=(n // BN,),
        in_specs=[pl.BlockSpec((BN, K), lambda i: (i, 0)),
                  pl.BlockSpec((K, Do), lambda i: (0, 0))],
        out_specs=pl.BlockSpec((BN, Do), lambda i: (i, 0)),
        out_shape=jax.ShapeDtypeStruct((n, Do), jnp.float32),
    )(x, W)


def _combine_matmul(y, p, b, eps, W):
    n, D = y.shape
    Do = W.shape[1]
    BN = 1000

    def body(e_ref, y_ref, p0_ref, p1_ref, b_ref, w_ref, o_ref):
        s = 1.0 + e_ref[0, 0]
        h = jnp.maximum(s * y_ref[...] + p0_ref[0] + p1_ref[0]
                        + b_ref[...], 0.0)
        o_ref[...] = jnp.dot(h, w_ref[...], preferred_element_type=jnp.float32)

    return pl.pallas_call(
        body,
        grid=(n // BN,),
        in_specs=[pl.BlockSpec(memory_space=pltpu.SMEM),
                  pl.BlockSpec((BN, D), lambda i: (i, 0)),
                  pl.BlockSpec((1, BN, D), lambda i: (0, i, 0)),
                  pl.BlockSpec((1, BN, D), lambda i: (1, i, 0)),
                  pl.BlockSpec((1, D), lambda i: (0, 0)),
                  pl.BlockSpec((D, Do), lambda i: (0, 0))],
        out_specs=pl.BlockSpec((BN, Do), lambda i: (i, 0)),
        out_shape=jax.ShapeDtypeStruct((n, Do), jnp.float32),
    )(eps.reshape(1, 1), y, p, p, b.reshape(1, D), W)


def _combine_relu(y, p, b, eps):
    n, D = y.shape
    BN = 1000

    def body(e_ref, y_ref, p0_ref, p1_ref, b_ref, o_ref):
        s = 1.0 + e_ref[0, 0]
        o_ref[...] = jnp.maximum(s * y_ref[...] + p0_ref[0] + p1_ref[0]
                                 + b_ref[...], 0.0)

    return pl.pallas_call(
        body,
        grid=(n // BN,),
        in_specs=[pl.BlockSpec(memory_space=pltpu.SMEM),
                  pl.BlockSpec((BN, D), lambda i: (i, 0)),
                  pl.BlockSpec((1, BN, D), lambda i: (0, i, 0)),
                  pl.BlockSpec((1, BN, D), lambda i: (1, i, 0)),
                  pl.BlockSpec((1, D), lambda i: (0, 0))],
        out_specs=pl.BlockSpec((BN, D), lambda i: (i, 0)),
        out_shape=jax.ShapeDtypeStruct((n, D), jnp.float32),
    )(eps.reshape(1, 1), y, p, p, b.reshape(1, D))


def _final_softmax(h, p, W, b, eps):
    n, D = h.shape
    Do = W.shape[1]
    BN = 1000

    def body(e_ref, h_ref, p0_ref, p1_ref, w_ref, b_ref, lo_ref, pr_ref):
        s = 1.0 + e_ref[0, 0]
        z = s * h_ref[...] + p0_ref[0] + p1_ref[0]
        l = jnp.maximum(jnp.dot(z, w_ref[...],
                                preferred_element_type=jnp.float32)
                        + b_ref[...], 0.0)
        lo_ref[...] = l
        m = jnp.max(l, axis=-1, keepdims=True)
        ex = jnp.exp(l - m)
        pr_ref[...] = ex / jnp.sum(ex, axis=-1, keepdims=True)

    return pl.pallas_call(
        body,
        grid=(n // BN,),
        in_specs=[pl.BlockSpec(memory_space=pltpu.SMEM),
                  pl.BlockSpec((BN, D), lambda i: (i, 0)),
                  pl.BlockSpec((1, BN, D), lambda i: (0, i, 0)),
                  pl.BlockSpec((1, BN, D), lambda i: (1, i, 0)),
                  pl.BlockSpec((D, Do), lambda i: (0, 0)),
                  pl.BlockSpec((1, Do), lambda i: (0, 0))],
        out_specs=[pl.BlockSpec((BN, Do), lambda i: (i, 0)),
                   pl.BlockSpec((BN, Do), lambda i: (i, 0))],
        out_shape=[jax.ShapeDtypeStruct((n, Do), jnp.float32),
                   jax.ShapeDtypeStruct((n, Do), jnp.float32)],
    )(eps.reshape(1, 1), h, p, p, W, b.reshape(1, Do))


def kernel(x, edge_index, W1, b1, eps1, W2, b2, eps2, W3, b3, eps3):
    pad_src = jnp.zeros((EPAD,), jnp.int32)
    pad_dst = jnp.full((EPAD,), N, jnp.int32)
    src3 = jnp.concatenate([edge_index[0], pad_src]).reshape(NW, NCHUNK, CHUNK)
    dst3 = jnp.concatenate([edge_index[1], pad_dst]).reshape(NW, NCHUNK, CHUNK)
    z128 = jnp.zeros((RPS, 128), jnp.float32)

    y1 = _matmul(x, W1)
    p1p = _seg_sum_partials(y1, src3, dst3, z128)
    y2 = _combine_matmul(y1, p1p, b1, eps1, W2)
    p2p = _seg_sum_partials(y2, src3, dst3, z128)
    h2 = _combine_relu(y2, p2p, b2, eps2)
    p3p = _seg_sum_partials(h2, src3, dst3, z128)
    logits, probs = _final_softmax(h2, p3p, W3, b3, eps3)
    return (logits, probs)


# speedup vs baseline: 1.0581x; 1.0581x over previous
import functools

import jax
import jax.numpy as jnp
from jax import lax
from jax.experimental import pallas as pl
from jax.experimental.pallas import tpu as pltpu
from jax.experimental.pallas import tpu_sc as plsc

N = 10000
NPAD = 10240
E = 320000
NC = 2
NS = 16
NW = NC * NS
CHUNK = 80
NCHUNK = 128
HALF = NCHUNK // 2
EPW = NCHUNK * CHUNK
EPAD = NW * EPW - E
RPS = NPAD // NS


def _seg_sum_partials(y, src3, dst3, zrows):
    D = y.shape[1]
    mesh = plsc.VectorSubcoreMesh(core_axis_name="c", subcore_axis_name="s")

    @functools.partial(
        pl.kernel,
        out_type=jax.ShapeDtypeStruct((NC, NPAD, D), jnp.float32),
        mesh=mesh,
        scratch_types=[
            pltpu.VMEM((HALF, CHUNK), jnp.int32),
            pltpu.VMEM((HALF, CHUNK), jnp.int32),
            pltpu.VMEM((CHUNK, D), jnp.float32),
            pltpu.VMEM((CHUNK, D), jnp.float32),
            pltpu.VMEM_SHARED((NPAD, D), jnp.float32),
            pltpu.SemaphoreType.DMA,
            pltpu.SemaphoreType.DMA,
        ],
    )
    def k(y_hbm, src_hbm, dst_hbm, z_hbm, out_hbm,
          src_v, dst_v, rows0, rows1, acc, sem0, sem1):
        cid = lax.axis_index("c")
        sid = lax.axis_index("s")
        wid = cid * NS + sid

        pltpu.sync_copy(z_hbm, acc.at[pl.ds(sid * RPS, RPS)])
        plsc.subcore_barrier()

        def scatter(ci, buf):
            pltpu.sync_copy(buf, acc.at[dst_v.at[ci]], add=True)

        for h in range(2):
            pltpu.sync_copy(src_hbm.at[wid, pl.ds(h * HALF, HALF)], src_v)
            pltpu.sync_copy(dst_hbm.at[wid, pl.ds(h * HALF, HALF)], dst_v)

            @pl.loop(0, HALF, step=2)
            def _(ci):
                g0 = pltpu.async_copy(y_hbm.at[src_v.at[ci]], rows0, sem0)
                g1 = pltpu.async_copy(y_hbm.at[src_v.at[ci + 1]], rows1, sem1)
                g0.wait()
                scatter(ci, rows0)
                g1.wait()
                scatter(ci + 1, rows1)

        plsc.subcore_barrier()
        pltpu.sync_copy(acc.at[pl.ds(sid * RPS, RPS)],
                        out_hbm.at[cid, pl.ds(sid * RPS, RPS)])

    return k(y, src3, dst3, zrows)


def _matmul(x, W):
    n, K = x.shape
    Do = W.shape[1]
    BN = 1000

    def body(x_ref, w_ref, o_ref):
        o_ref[...] = jnp.dot(x_ref[...], w_ref[...],
                             preferred_element_type=jnp.float32)

    return pl.pallas_call(
        body,
        grid=(n // BN,),
        in_specs=[pl.BlockSpec((BN, K), lambda i: (i, 0)),
                  pl.BlockSpec((K, Do), lambda i: (0, 0))],
        out_specs=pl.BlockSpec((BN, Do), lambda i: (i, 0)),
        out_shape=jax.ShapeDtypeStruct((n, Do), jnp.float32),
    )(x, W)


def _combine_matmul(y, p, b, eps, W):
    n, D = y.shape
    Do = W.shape[1]
    BN = 1000

    def body(e_ref, y_ref, p0_ref, p1_ref, b_ref, w_ref, o_ref):
        s = 1.0 + e_ref[0, 0]
        h = jnp.maximum(s * y_ref[...] + p0_ref[0] + p1_ref[0]
                        + b_ref[...], 0.0)
        o_ref[...] = jnp.dot(h, w_ref[...], preferred_element_type=jnp.float32)

    return pl.pallas_call(
        body,
        grid=(n // BN,),
        in_specs=[pl.BlockSpec(memory_space=pltpu.SMEM),
                  pl.BlockSpec((BN, D), lambda i: (i, 0)),
                  pl.BlockSpec((1, BN, D), lambda i: (0, i, 0)),
                  pl.BlockSpec((1, BN, D), lambda i: (1, i, 0)),
                  pl.BlockSpec((1, D), lambda i: (0, 0)),
                  pl.BlockSpec((D, Do), lambda i: (0, 0))],
        out_specs=pl.BlockSpec((BN, Do), lambda i: (i, 0)),
        out_shape=jax.ShapeDtypeStruct((n, Do), jnp.float32),
    )(eps.reshape(1, 1), y, p, p, b.reshape(1, D), W)


def _combine_relu(y, p, b, eps):
    n, D = y.shape
    BN = 1000

    def body(e_ref, y_ref, p0_ref, p1_ref, b_ref, o_ref):
        s = 1.0 + e_ref[0, 0]
        o_ref[...] = jnp.maximum(s * y_ref[...] + p0_ref[0] + p1_ref[0]
                                 + b_ref[...], 0.0)

    return pl.pallas_call(
        body,
        grid=(n // BN,),
        in_specs=[pl.BlockSpec(memory_space=pltpu.SMEM),
                  pl.BlockSpec((BN, D), lambda i: (i, 0)),
                  pl.BlockSpec((1, BN, D), lambda i: (0, i, 0)),
                  pl.BlockSpec((1, BN, D), lambda i: (1, i, 0)),
                  pl.BlockSpec((1, D), lambda i: (0, 0))],
        out_specs=pl.BlockSpec((BN, D), lambda i: (i, 0)),
        out_shape=jax.ShapeDtypeStruct((n, D), jnp.float32),
    )(eps.reshape(1, 1), y, p, p, b.reshape(1, D))


def _final_softmax(h, p, W, b, eps):
    n, D = h.shape
    Do = W.shape[1]
    BN = 1000

    def body(e_ref, h_ref, p0_ref, p1_ref, w_ref, b_ref, lo_ref, pr_ref):
        s = 1.0 + e_ref[0, 0]
        z = s * h_ref[...] + p0_ref[0] + p1_ref[0]
        l = jnp.maximum(jnp.dot(z, w_ref[...],
                                preferred_element_type=jnp.float32)
                        + b_ref[...], 0.0)
        lo_ref[...] = l
        m = jnp.max(l, axis=-1, keepdims=True)
        ex = jnp.exp(l - m)
        pr_ref[...] = ex / jnp.sum(ex, axis=-1, keepdims=True)

    return pl.pallas_call(
        body,
        grid=(n // BN,),
        in_specs=[pl.BlockSpec(memory_space=pltpu.SMEM),
                  pl.BlockSpec((BN, D), lambda i: (i, 0)),
                  pl.BlockSpec((1, BN, D), lambda i: (0, i, 0)),
                  pl.BlockSpec((1, BN, D), lambda i: (1, i, 0)),
                  pl.BlockSpec((D, Do), lambda i: (0, 0)),
                  pl.BlockSpec((1, Do), lambda i: (0, 0))],
        out_specs=[pl.BlockSpec((BN, Do), lambda i: (i, 0)),
                   pl.BlockSpec((BN, Do), lambda i: (i, 0))],
        out_shape=[jax.ShapeDtypeStruct((n, Do), jnp.float32),
                   jax.ShapeDtypeStruct((n, Do), jnp.float32)],
    )(eps.reshape(1, 1), h, p, p, W, b.reshape(1, Do))


def kernel(x, edge_index, W1, b1, eps1, W2, b2, eps2, W3, b3, eps3):
    pad_src = jnp.zeros((EPAD,), jnp.int32)
    pad_dst = jnp.full((EPAD,), N, jnp.int32)
    src3 = jnp.concatenate([edge_index[0], pad_src]).reshape(NW, NCHUNK, CHUNK)
    dst3 = jnp.concatenate([edge_index[1], pad_dst]).reshape(NW, NCHUNK, CHUNK)
    z128 = jnp.zeros((RPS, 128), jnp.float32)

    y1 = _matmul(x, W1)
    p1p = _seg_sum_partials(y1, src3, dst3, z128)
    y2 = _combine_matmul(y1, p1p, b1, eps1, W2)
    p2p = _seg_sum_partials(y2, src3, dst3, z128)
    h2 = _combine_relu(y2, p2p, b2, eps2)
    p3p = _seg_sum_partials(h2, src3, dst3, z128)
    logits, probs = _final_softmax(h2, p3p, W3, b3, eps3)
    return (logits, probs)


# speedup vs baseline: 2.2213x; 2.0993x over previous
import functools

import jax
import jax.numpy as jnp
from jax import lax
from jax.experimental import pallas as pl
from jax.experimental.pallas import tpu as pltpu
from jax.experimental.pallas import tpu_sc as plsc

N = 10000
NPAD = 10240
E = 320000
NC = 2
NS = 16
NW = NC * NS
EPW = E // NW
CHUNK = 80
NCHUNK = EPW // CHUNK
RPS = NPAD // NS


def _seg_sum_partials(y, src3, dst3, zrows):
    D = y.shape[1]
    mesh = plsc.VectorSubcoreMesh(core_axis_name="c", subcore_axis_name="s")

    @functools.partial(
        pl.kernel,
        out_type=jax.ShapeDtypeStruct((NC, NPAD, D), jnp.float32),
        mesh=mesh,
        scratch_types=[
            pltpu.VMEM((NCHUNK, CHUNK), jnp.int32),
            pltpu.VMEM((NCHUNK, CHUNK), jnp.int32),
            pltpu.VMEM((CHUNK, D), jnp.float32),
            pltpu.VMEM_SHARED((NPAD, D), jnp.float32),
        ],
    )
    def k(y_hbm, src_hbm, dst_hbm, z_hbm, out_hbm, src_v, dst_v, rows_v, acc):
        cid = lax.axis_index("c")
        sid = lax.axis_index("s")
        wid = cid * NS + sid

        pltpu.sync_copy(z_hbm, acc.at[pl.ds(sid * RPS, RPS)])
        pltpu.sync_copy(src_hbm.at[wid], src_v)
        pltpu.sync_copy(dst_hbm.at[wid], dst_v)
        plsc.subcore_barrier()

        @pl.loop(0, NCHUNK)
        def _(ci):
            pltpu.sync_copy(y_hbm.at[src_v.at[ci]], rows_v)
            pltpu.sync_copy(rows_v, acc.at[dst_v.at[ci]], add=True)

        plsc.subcore_barrier()
        pltpu.sync_copy(acc.at[pl.ds(sid * RPS, RPS)],
                        out_hbm.at[cid, pl.ds(sid * RPS, RPS)])

    return k(y, src3, dst3, zrows)


def _matmul(x, W):
    n, K = x.shape
    Do = W.shape[1]
    BN = 1000

    def body(x_ref, w_ref, o_ref):
        o_ref[...] = jnp.dot(x_ref[...], w_ref[...],
                             preferred_element_type=jnp.float32)

    return pl.pallas_call(
        body,
        grid=(n // BN,),
        in_specs=[pl.BlockSpec((BN, K), lambda i: (i, 0)),
                  pl.BlockSpec((K, Do), lambda i: (0, 0))],
        out_specs=pl.BlockSpec((BN, Do), lambda i: (i, 0)),
        out_shape=jax.ShapeDtypeStruct((n, Do), jnp.float32),
    )(x, W)


def _combine_matmul(y, p, b, eps, W):
    n, D = y.shape
    Do = W.shape[1]
    BN = 1000

    def body(e_ref, y_ref, p0_ref, p1_ref, w_ref, b_ref, o_ref):
        s = 1.0 + e_ref[0, 0]
        z = s * y_ref[...] + p0_ref[0] + p1_ref[0]
        o_ref[...] = jnp.maximum(
            jnp.dot(z, w_ref[...], preferred_element_type=jnp.float32)
            + b_ref[...], 0.0)

    return pl.pallas_call(
        body,
        grid=(n // BN,),
        in_specs=[pl.BlockSpec(memory_space=pltpu.SMEM),
                  pl.BlockSpec((BN, D), lambda i: (i, 0)),
                  pl.BlockSpec((1, BN, D), lambda i: (0, i, 0)),
                  pl.BlockSpec((1, BN, D), lambda i: (1, i, 0)),
                  pl.BlockSpec((D, Do), lambda i: (0, 0)),
                  pl.BlockSpec((1, Do), lambda i: (0, 0))],
        out_specs=pl.BlockSpec((BN, Do), lambda i: (i, 0)),
        out_shape=jax.ShapeDtypeStruct((n, Do), jnp.float32),
    )(eps.reshape(1, 1), y, p, p, W, b.reshape(1, Do))


def _combine_relu(y, p, b, eps):
    n, D = y.shape
    BN = 1000

    def body(e_ref, y_ref, p0_ref, p1_ref, b_ref, o_ref):
        s = 1.0 + e_ref[0, 0]
        o_ref[...] = jnp.maximum(s * y_ref[...] + p0_ref[0] + p1_ref[0]
                                 + b_ref[...], 0.0)

    return pl.pallas_call(
        body,
        grid=(n // BN,),
        in_specs=[pl.BlockSpec(memory_space=pltpu.SMEM),
                  pl.BlockSpec((BN, D), lambda i: (i, 0)),
                  pl.BlockSpec((1, BN, D), lambda i: (0, i, 0)),
                  pl.BlockSpec((1, BN, D), lambda i: (1, i, 0)),
                  pl.BlockSpec((1, D), lambda i: (0, 0))],
        out_specs=pl.BlockSpec((BN, D), lambda i: (i, 0)),
        out_shape=jax.ShapeDtypeStruct((n, D), jnp.float32),
    )(eps.reshape(1, 1), y, p, p, b.reshape(1, D))


def _final_softmax(h, p, W, b, eps):
    n, D = h.shape
    Do = W.shape[1]
    BN = 1000

    def body(e_ref, h_ref, p0_ref, p1_ref, w_ref, b_ref, lo_ref, pr_ref):
        s = 1.0 + e_ref[0, 0]
        z = s * h_ref[...] + p0_ref[0] + p1_ref[0]
        l = jnp.maximum(jnp.dot(z, w_ref[...],
                                preferred_element_type=jnp.float32)
                        + b_ref[...], 0.0)
        lo_ref[...] = l
        m = jnp.max(l, axis=-1, keepdims=True)
        ex = jnp.exp(l - m)
        pr_ref[...] = ex / jnp.sum(ex, axis=-1, keepdims=True)

    return pl.pallas_call(
        body,
        grid=(n // BN,),
        in_specs=[pl.BlockSpec(memory_space=pltpu.SMEM),
                  pl.BlockSpec((BN, D), lambda i: (i, 0)),
                  pl.BlockSpec((1, BN, D), lambda i: (0, i, 0)),
                  pl.BlockSpec((1, BN, D), lambda i: (1, i, 0)),
                  pl.BlockSpec((D, Do), lambda i: (0, 0)),
                  pl.BlockSpec((1, Do), lambda i: (0, 0))],
        out_specs=[pl.BlockSpec((BN, Do), lambda i: (i, 0)),
                   pl.BlockSpec((BN, Do), lambda i: (i, 0))],
        out_shape=[jax.ShapeDtypeStruct((n, Do), jnp.float32),
                   jax.ShapeDtypeStruct((n, Do), jnp.float32)],
    )(eps.reshape(1, 1), h, p, p, W, b.reshape(1, Do))


def kernel(x, edge_index, W1, b1, eps1, W2, b2, eps2, W3, b3, eps3):
    src3 = edge_index[0].reshape(NW, NCHUNK, CHUNK)
    dst3 = edge_index[1].reshape(NW, NCHUNK, CHUNK)
    z128 = jnp.zeros((RPS, 128), jnp.float32)

    p1 = _seg_sum_partials(x, src3, dst3, z128)
    h1 = _combine_matmul(x, p1, b1, eps1, W1)
    p2 = _seg_sum_partials(h1, src3, dst3, z128)
    h2 = _combine_matmul(h1, p2, b2, eps2, W2)
    p3 = _seg_sum_partials(h2, src3, dst3, z128)
    logits, probs = _final_softmax(h2, p3, W3, b3, eps3)
    return (logits, probs)
